# Initial kernel scaffold; baseline (speedup 1.0000x reference)
import jax
import jax.numpy as jnp
from jax.experimental import pallas as pl


def kernel(x, params):
    raise NotImplementedError("write your pallas kernel here")


# speedup vs baseline: 1.6452x; 1.6452x over previous
import functools

import jax
import jax.numpy as jnp
import numpy as np
from jax.experimental import pallas as pl
from jax.experimental.pallas import tpu as pltpu

DIM = 96
HEADS = 3
HD = DIM // HEADS
NWIN = 7
P2 = NWIN * NWIN
WS = 8
HW = WS * WS
NPIX = P2 * HW
TOPK = 4
MLPR = 4
SDW = 5
SCALE = DIM ** -0.5
F32 = jnp.float32


def _win_to_img(t):
    return (t.reshape(NWIN, NWIN, WS, WS, DIM)
             .transpose(0, 2, 1, 3, 4)
             .reshape(NWIN * WS, NWIN * WS, DIM))


def _img_to_win(t):
    return (t.reshape(NWIN, WS, NWIN, WS, DIM)
             .transpose(0, 2, 1, 3, 4)
             .reshape(NPIX, DIM))


def _layernorm(xf, g, b):
    mu = jnp.mean(xf, axis=-1, keepdims=True)
    xc = xf - mu
    var = jnp.mean(xc * xc, axis=-1, keepdims=True)
    return xc * jax.lax.rsqrt(var + 1e-6) * g + b


def _pre_kernel(x_ref, g1_ref, b1_ref, wq_ref, wk_ref, wv_ref, bq_ref,
                bk_ref, bv_ref, wl_ref, bl_ref,
                q_ref, k_ref, v_ref, lepe_ref, topi_ref, pad_ref):
    xf = x_ref[0]
    y = _layernorm(xf, g1_ref[...], b1_ref[...])
    q = jnp.dot(y, wq_ref[...], preferred_element_type=F32) + bq_ref[...]
    k = jnp.dot(y, wk_ref[...], preferred_element_type=F32) + bk_ref[...]
    v = jnp.dot(y, wv_ref[...], preferred_element_type=F32) + bv_ref[...]
    q_ref[0] = q.reshape(P2, HW, DIM)
    k_ref[0] = k.reshape(P2, HW, DIM)
    v_ref[0] = v.reshape(P2, HW, DIM)

    pad_ref[...] = jnp.zeros((60, 60, DIM), F32)
    pad_ref[2:58, 2:58, :] = _win_to_img(v)
    acc = jnp.broadcast_to(bl_ref[...], (NPIX, DIM))
    for di in range(SDW):
        for dj in range(SDW):
            sh = pad_ref[di:di + 56, dj:dj + 56, :].reshape(NPIX, DIM)
            acc = acc + sh * wl_ref[di * SDW + dj:di * SDW + dj + 1, :]
    lepe_ref[0] = _img_to_win(acc.reshape(56, 56, DIM)).reshape(NPIX, DIM)

    qm = jnp.mean(q.reshape(P2, HW, DIM), axis=1)
    km = jnp.mean(k.reshape(P2, HW, DIM), axis=1)
    lT = jax.lax.dot_general(km, qm * SCALE, (((1,), (1,)), ((), ())),
                             preferred_element_type=F32)
    iota0 = jax.lax.broadcasted_iota(jnp.int32, (P2, P2), 0)
    for t in range(TOPK):
        mx = jnp.max(lT, axis=0, keepdims=True)
        cand = jnp.where(lT >= mx, iota0, jnp.int32(2 ** 30))
        idx = jnp.min(cand, axis=0, keepdims=True)
        topi_ref[0, t:t + 1, :] = idx
        lT = jnp.where(iota0 == idx, -jnp.inf, lT)


def _attn_kernel(topi_sm, q_ref, k_ref, v_ref, lepe_ref, x_ref,
                 wo_ref, bo_ref, g2_ref, b2_ref, w1_ref, bm1_ref,
                 w2_ref, bm2_ref, out_ref, ao_ref):
    n = pl.program_id(0)

    def body(p, carry):
        qwin = q_ref[0, p] * SCALE
        ks = []
        vs = []
        for t in range(TOPK):
            s = topi_sm[n, t, p]
            ks.append(k_ref[0, s])
            vs.append(v_ref[0, s])
        kk = jnp.concatenate(ks, axis=0)
        vv = jnp.concatenate(vs, axis=0)
        outs = []
        for hh in range(HEADS):
            qh = qwin[:, hh * HD:(hh + 1) * HD]
            kh = kk[:, hh * HD:(hh + 1) * HD]
            s = jax.lax.dot_general(qh, kh, (((1,), (1,)), ((), ())),
                                    preferred_element_type=F32)
            s = s - jnp.max(s, axis=1, keepdims=True)
            e = jnp.exp(s)
            a = e / jnp.sum(e, axis=1, keepdims=True)
            outs.append(jnp.dot(a, vv[:, hh * HD:(hh + 1) * HD],
                                preferred_element_type=F32))
        ao_ref[p] = jnp.concatenate(outs, axis=1)
        return carry

    jax.lax.fori_loop(0, P2, body, 0)

    ao = ao_ref[...].reshape(NPIX, DIM) + lepe_ref[0]
    x1 = x_ref[0] + jnp.dot(ao, wo_ref[...],
                            preferred_element_type=F32) + bo_ref[...]
    y = _layernorm(x1, g2_ref[...], b2_ref[...])
    t1 = jnp.dot(y, w1_ref[...], preferred_element_type=F32) + bm1_ref[...]
    t1 = t1 * 0.5 * (1.0 + jax.lax.erf(t1 * (2.0 ** -0.5)))
    y2 = jnp.dot(t1, w2_ref[...], preferred_element_type=F32) + bm2_ref[...]
    out_ref[0] = x1 + y2


def _tail_kernel(x_ref, cw1_ref, cb1_ref, cw2_ref, cb2_ref, sw_ref, sb_ref,
                 wc_ref, bc_ref, out_ref, pad_ref):
    xf = x_ref[0]
    xm = jnp.mean(xf, axis=0, keepdims=True)
    c1 = jax.nn.relu(jnp.dot(xm, cw1_ref[...],
                             preferred_element_type=F32) + cb1_ref[...])
    cse = jax.nn.sigmoid(jnp.dot(c1, cw2_ref[...],
                                 preferred_element_type=F32) + cb2_ref[...])
    sse = jax.nn.sigmoid(jnp.dot(xf, sw_ref[...],
                                 preferred_element_type=F32) + sb_ref[...])
    y = xf * (cse + sse)

    pad_ref[...] = jnp.zeros((58, 58, DIM), F32)
    pad_ref[1:57, 1:57, :] = _win_to_img(y)
    acc = jnp.broadcast_to(bc_ref[...], (NPIX, DIM))
    for di in range(3):
        for dj in range(3):
            sh = pad_ref[di:di + 56, dj:dj + 56, :].reshape(NPIX, DIM)
            acc = acc + jnp.dot(sh, wc_ref[di * 3 + dj],
                                preferred_element_type=F32)
    out_ref[0] = acc.reshape(56, 56, DIM)


def _full(shape):
    nd = len(shape)
    return pl.BlockSpec(shape, lambda n, *_: (0,) * nd)


def _per_img(shape):
    nd = len(shape)
    return pl.BlockSpec((1,) + shape, lambda n, *_: (n,) + (0,) * nd)


def _row2(a):
    return a.reshape(1, -1)


@jax.jit
def kernel(x, params):
    N = x.shape[0]
    xw = jnp.transpose(x, (0, 2, 3, 1))
    xw = (xw.reshape(N, NWIN, WS, NWIN, WS, DIM)
            .transpose(0, 1, 3, 2, 4, 5)
            .reshape(N, NPIX, DIM))

    for p in params['blocks']:
        wq = p['Wqkv'][:, :DIM]
        wk = p['Wqkv'][:, DIM:2 * DIM]
        wv = p['Wqkv'][:, 2 * DIM:]
        bq = _row2(p['bqkv'][:DIM])
        bk = _row2(p['bqkv'][DIM:2 * DIM])
        bv = _row2(p['bqkv'][2 * DIM:])
        wl = p['Wlepe'].reshape(SDW * SDW, DIM)

        q, k, v, lepe, topi = pl.pallas_call(
            _pre_kernel,
            grid=(N,),
            in_specs=[_per_img((NPIX, DIM))] + [
                _full(s.shape) for s in
                (_row2(p['g1']), _row2(p['b1']), wq, wk, wv, bq, bk, bv,
                 wl, _row2(p['blepe']))],
            out_specs=[_per_img((P2, HW, DIM))] * 3 + [
                _per_img((NPIX, DIM)),
                _per_img((TOPK, P2))],
            out_shape=[jax.ShapeDtypeStruct((N, P2, HW, DIM), F32)] * 3 + [
                jax.ShapeDtypeStruct((N, NPIX, DIM), F32),
                jax.ShapeDtypeStruct((N, TOPK, P2), jnp.int32)],
            scratch_shapes=[pltpu.VMEM((60, 60, DIM), F32)],
        )(xw, _row2(p['g1']), _row2(p['b1']), wq, wk, wv, bq, bk, bv,
          wl, _row2(p['blepe']))

        xw = pl.pallas_call(
            _attn_kernel,
            grid_spec=pltpu.PrefetchScalarGridSpec(
                num_scalar_prefetch=1,
                grid=(N,),
                in_specs=[_per_img((P2, HW, DIM))] * 3 + [
                    _per_img((NPIX, DIM))] * 2 + [
                    _full(s.shape) for s in
                    (p['Wo'], _row2(p['bo']), _row2(p['g2']), _row2(p['b2']),
                     p['W1'], _row2(p['bm1']), p['W2'], _row2(p['bm2']))],
                out_specs=_per_img((NPIX, DIM)),
                scratch_shapes=[pltpu.VMEM((P2, HW, DIM), F32)],
            ),
            out_shape=jax.ShapeDtypeStruct((N, NPIX, DIM), F32),
        )(topi, q, k, v, lepe, xw,
          p['Wo'], _row2(p['bo']), _row2(p['g2']), _row2(p['b2']),
          p['W1'], _row2(p['bm1']), p['W2'], _row2(p['bm2']))

    out = pl.pallas_call(
        _tail_kernel,
        grid=(N,),
        in_specs=[_per_img((NPIX, DIM))] + [
            _full(s.shape) for s in
            (params['cse_w1'], _row2(params['cse_b1']),
             params['cse_w2'], _row2(params['cse_b2']),
             params['sse_w'], _row2(params['sse_b']),
             params['conv_w'].reshape(9, DIM, DIM), _row2(params['conv_b']))],
        out_specs=_per_img((56, 56, DIM)),
        out_shape=jax.ShapeDtypeStruct((N, 56, 56, DIM), F32),
        scratch_shapes=[pltpu.VMEM((58, 58, DIM), F32)],
    )(xw, params['cse_w1'], _row2(params['cse_b1']),
      params['cse_w2'], _row2(params['cse_b2']),
      params['sse_w'], _row2(params['sse_b']),
      params['conv_w'].reshape(9, DIM, DIM), _row2(params['conv_b']))

    return jnp.transpose(out, (0, 3, 1, 2))


# speedup vs baseline: 2.0940x; 1.2728x over previous
import functools

import jax
import jax.numpy as jnp
import numpy as np
from jax.experimental import pallas as pl
from jax.experimental.pallas import tpu as pltpu

DIM = 96
HEADS = 3
HD = DIM // HEADS
NWIN = 7
P2 = NWIN * NWIN
WS = 8
HW = WS * WS
NPIX = P2 * HW
TOPK = 4
MLPR = 4
SDW = 5
SCALE = DIM ** -0.5
F32 = jnp.float32
BF16 = jnp.bfloat16


def _win_to_img(t):
    return (t.reshape(NWIN, NWIN, WS, WS, DIM)
             .transpose(0, 2, 1, 3, 4)
             .reshape(NWIN * WS, NWIN * WS, DIM))


def _img_to_win(t):
    return (t.reshape(NWIN, WS, NWIN, WS, DIM)
             .transpose(0, 2, 1, 3, 4)
             .reshape(NPIX, DIM))


def _layernorm(xf, g, b):
    mu = jnp.mean(xf, axis=-1, keepdims=True)
    xc = xf - mu
    var = jnp.mean(xc * xc, axis=-1, keepdims=True)
    return xc * jax.lax.rsqrt(var + 1e-6) * g + b


def _bdot(a, b):
    return jnp.dot(a.astype(BF16), b, preferred_element_type=F32)


def _pre_body(xf, g1_ref, b1_ref, wq_ref, wk_ref, wv_ref, bq_ref, bk_ref,
              bv_ref, wl_ref, bl_ref,
              q_ref, k_ref, v_ref, lepe_ref, topi_ref, pad_ref):
    y = _layernorm(xf, g1_ref[...], b1_ref[...])
    q = jnp.dot(y, wq_ref[...], preferred_element_type=F32) + bq_ref[...]
    k = jnp.dot(y, wk_ref[...], preferred_element_type=F32) + bk_ref[...]
    v = jnp.dot(y, wv_ref[...], preferred_element_type=F32) + bv_ref[...]
    q_ref[0] = q.astype(BF16).reshape(P2, HW, DIM)
    k_ref[0] = k.astype(BF16).reshape(P2, HW, DIM)
    v_ref[0] = v.astype(BF16).reshape(P2, HW, DIM)

    pad_ref[...] = jnp.zeros((60, 60, DIM), F32)
    pad_ref[2:58, 2:58, :] = _win_to_img(v)
    acc = jnp.broadcast_to(bl_ref[...], (NPIX, DIM))
    for di in range(SDW):
        for dj in range(SDW):
            sh = pad_ref[di:di + 56, dj:dj + 56, :].reshape(NPIX, DIM)
            acc = acc + sh * wl_ref[di * SDW + dj:di * SDW + dj + 1, :]
    lepe_ref[0] = _img_to_win(acc.reshape(56, 56, DIM)).reshape(NPIX, DIM)

    qm = jnp.mean(q.reshape(P2, HW, DIM), axis=1)
    km = jnp.mean(k.reshape(P2, HW, DIM), axis=1)
    lT = jax.lax.dot_general(km, qm * SCALE, (((1,), (1,)), ((), ())),
                             preferred_element_type=F32)
    iota0 = jax.lax.broadcasted_iota(jnp.int32, (P2, P2), 0)
    for t in range(TOPK):
        mx = jnp.max(lT, axis=0, keepdims=True)
        cand = jnp.where(lT >= mx, iota0, jnp.int32(2 ** 30))
        idx = jnp.min(cand, axis=0, keepdims=True)
        topi_ref[0, t:t + 1, :] = idx
        lT = jnp.where(iota0 == idx, -jnp.inf, lT)


def _post_body(ao_ref, lepe_ref, x_ref, wo_ref, bo_ref, g2_ref, b2_ref,
               w1_ref, bm1_ref, w2_ref, bm2_ref):
    ao = ao_ref[0] + lepe_ref[0]
    x1 = x_ref[0] + _bdot(ao, wo_ref[...]) + bo_ref[...]
    y = _layernorm(x1, g2_ref[...], b2_ref[...])
    t1 = _bdot(y, w1_ref[...]) + bm1_ref[...]
    t1 = t1 * 0.5 * (1.0 + jax.lax.erf(t1 * (2.0 ** -0.5)))
    y2 = _bdot(t1, w2_ref[...]) + bm2_ref[...]
    return x1 + y2


def _pre_kernel(x_ref, *refs):
    _pre_body(x_ref[0], *refs)


def _mid_kernel(ao_ref, lepe_ref, x_ref, wo_ref, bo_ref, g2_ref, b2_ref,
                w1_ref, bm1_ref, w2_ref, bm2_ref,
                g1_ref, b1_ref, wq_ref, wk_ref, wv_ref, bq_ref, bk_ref,
                bv_ref, wl_ref, bl_ref,
                xo_ref, q_ref, k_ref, v_ref, lepe2_ref, topi_ref, pad_ref):
    x2 = _post_body(ao_ref, lepe_ref, x_ref, wo_ref, bo_ref, g2_ref, b2_ref,
                    w1_ref, bm1_ref, w2_ref, bm2_ref)
    xo_ref[0] = x2
    _pre_body(x2, g1_ref, b1_ref, wq_ref, wk_ref, wv_ref, bq_ref, bk_ref,
              bv_ref, wl_ref, bl_ref,
              q_ref, k_ref, v_ref, lepe2_ref, topi_ref, pad_ref)


def _attn_kernel(topi_sm, q_ref, k0_ref, k1_ref, k2_ref, k3_ref,
                 v0_ref, v1_ref, v2_ref, v3_ref, ao_ref):
    q = q_ref[0, 0]
    kk = jnp.concatenate([k0_ref[0, 0], k1_ref[0, 0],
                          k2_ref[0, 0], k3_ref[0, 0]], axis=0)
    vv = jnp.concatenate([v0_ref[0, 0], v1_ref[0, 0],
                          v2_ref[0, 0], v3_ref[0, 0]], axis=0)
    outs = []
    for hh in range(HEADS):
        qh = q[:, hh * HD:(hh + 1) * HD]
        kh = kk[:, hh * HD:(hh + 1) * HD]
        s = jax.lax.dot_general(qh, kh, (((1,), (1,)), ((), ())),
                                preferred_element_type=F32)
        e = jnp.exp(s * SCALE)
        l = jnp.sum(e, axis=1, keepdims=True)
        o = _bdot(e, vv[:, hh * HD:(hh + 1) * HD])
        outs.append(o / l)
    ao_ref[0, 0] = jnp.concatenate(outs, axis=1)


def _tail_kernel(ao_ref, lepe_ref, x_ref, wo_ref, bo_ref, g2_ref, b2_ref,
                 w1_ref, bm1_ref, w2_ref, bm2_ref,
                 cw1_ref, cb1_ref, cw2_ref, cb2_ref, sw_ref, sb_ref,
                 wc_ref, bc_ref, out_ref, pad_ref):
    xf = _post_body(ao_ref, lepe_ref, x_ref, wo_ref, bo_ref, g2_ref, b2_ref,
                    w1_ref, bm1_ref, w2_ref, bm2_ref)
    xm = jnp.mean(xf, axis=0, keepdims=True)
    c1 = jax.nn.relu(jnp.dot(xm, cw1_ref[...],
                             preferred_element_type=F32) + cb1_ref[...])
    cse = jax.nn.sigmoid(jnp.dot(c1, cw2_ref[...],
                                 preferred_element_type=F32) + cb2_ref[...])
    sse = jax.nn.sigmoid(_bdot(xf, sw_ref[...]) + sb_ref[...])
    y = xf * (cse + sse)

    pad_ref[...] = jnp.zeros((58, 58, DIM), F32)
    pad_ref[1:57, 1:57, :] = _win_to_img(y)
    acc = jnp.broadcast_to(bc_ref[...], (NPIX, DIM))
    for di in range(3):
        for dj in range(3):
            sh = pad_ref[di:di + 56, dj:dj + 56, :].reshape(NPIX, DIM)
            acc = acc + _bdot(sh, wc_ref[di * 3 + dj])
    out_ref[0] = acc.reshape(56, 56, DIM)


def _full(shape):
    nd = len(shape)
    return pl.BlockSpec(shape, lambda n, *_: (0,) * nd)


def _per_img(shape):
    nd = len(shape)
    return pl.BlockSpec((1,) + shape, lambda n, *_: (n,) + (0,) * nd)


def _row2(a):
    return a.reshape(1, -1)


def _block_weights(p):
    wq = p['Wqkv'][:, :DIM]
    wk = p['Wqkv'][:, DIM:2 * DIM]
    wv = p['Wqkv'][:, 2 * DIM:]
    pre_args = (_row2(p['g1']), _row2(p['b1']), wq, wk, wv,
                _row2(p['bqkv'][:DIM]), _row2(p['bqkv'][DIM:2 * DIM]),
                _row2(p['bqkv'][2 * DIM:]),
                p['Wlepe'].reshape(SDW * SDW, DIM), _row2(p['blepe']))
    post_args = (p['Wo'].astype(BF16), _row2(p['bo']), _row2(p['g2']),
                 _row2(p['b2']), p['W1'].astype(BF16), _row2(p['bm1']),
                 p['W2'].astype(BF16), _row2(p['bm2']))
    return pre_args, post_args


def _pre_out(N):
    specs = ([_per_img((P2, HW, DIM))] * 3 +
             [_per_img((NPIX, DIM)), _per_img((TOPK, P2))])
    shapes = ([jax.ShapeDtypeStruct((N, P2, HW, DIM), BF16)] * 3 +
              [jax.ShapeDtypeStruct((N, NPIX, DIM), F32),
               jax.ShapeDtypeStruct((N, TOPK, P2), jnp.int32)])
    return specs, shapes


def _attention(N, topi, q, k, v):
    def _qspec():
        return pl.BlockSpec((1, 1, HW, DIM), lambda n, p, topi_sm: (n, p, 0, 0))

    def _gspec(t):
        return pl.BlockSpec(
            (1, 1, HW, DIM),
            lambda n, p, topi_sm, t=t: (n, topi_sm[n, t, p], 0, 0))

    return pl.pallas_call(
        _attn_kernel,
        grid_spec=pltpu.PrefetchScalarGridSpec(
            num_scalar_prefetch=1,
            grid=(N, P2),
            in_specs=[_qspec()] + [_gspec(t) for t in range(TOPK)] * 2,
            out_specs=pl.BlockSpec((1, 1, HW, DIM),
                                   lambda n, p, topi_sm: (n, p, 0, 0)),
        ),
        out_shape=jax.ShapeDtypeStruct((N, P2, HW, DIM), F32),
    )(topi, q, k, k, k, k, v, v, v, v)


@jax.jit
def kernel(x, params):
    N = x.shape[0]
    xw = jnp.transpose(x, (0, 2, 3, 1))
    xw = (xw.reshape(N, NWIN, WS, NWIN, WS, DIM)
            .transpose(0, 1, 3, 2, 4, 5)
            .reshape(N, NPIX, DIM))

    b0, b1 = params['blocks']
    pre0, post0 = _block_weights(b0)
    pre1, post1 = _block_weights(b1)

    pre_specs, pre_shapes = _pre_out(N)

    q, k, v, lepe, topi = pl.pallas_call(
        _pre_kernel,
        grid=(N,),
        in_specs=[_per_img((NPIX, DIM))] + [_full(a.shape) for a in pre0],
        out_specs=pre_specs,
        out_shape=pre_shapes,
        scratch_shapes=[pltpu.VMEM((60, 60, DIM), F32)],
    )(xw, *pre0)

    ao = _attention(N, topi, q, k, v)

    mid_in = ((ao.reshape(N, NPIX, DIM), lepe, xw) + post0 + pre1)
    xw2, q, k, v, lepe, topi = pl.pallas_call(
        _mid_kernel,
        grid=(N,),
        in_specs=[_per_img((NPIX, DIM))] * 3 +
                 [_full(a.shape) for a in post0 + pre1],
        out_specs=[_per_img((NPIX, DIM))] + pre_specs,
        out_shape=[jax.ShapeDtypeStruct((N, NPIX, DIM), F32)] + pre_shapes,
        scratch_shapes=[pltpu.VMEM((60, 60, DIM), F32)],
    )(*mid_in)

    ao = _attention(N, topi, q, k, v)

    tail_w = (params['cse_w1'], _row2(params['cse_b1']),
              params['cse_w2'], _row2(params['cse_b2']),
              params['sse_w'].astype(BF16), _row2(params['sse_b']),
              params['conv_w'].reshape(9, DIM, DIM).astype(BF16),
              _row2(params['conv_b']))
    out = pl.pallas_call(
        _tail_kernel,
        grid=(N,),
        in_specs=[_per_img((NPIX, DIM))] * 3 +
                 [_full(a.shape) for a in post1 + tail_w],
        out_specs=_per_img((56, 56, DIM)),
        out_shape=jax.ShapeDtypeStruct((N, 56, 56, DIM), F32),
        scratch_shapes=[pltpu.VMEM((58, 58, DIM), F32)],
    )(ao.reshape(N, NPIX, DIM), lepe, xw2, *(post1 + tail_w))

    return jnp.transpose(out, (0, 3, 1, 2))


# speedup vs baseline: 2.9834x; 1.4248x over previous
import functools

import jax
import jax.numpy as jnp
import numpy as np
from jax.experimental import pallas as pl
from jax.experimental.pallas import tpu as pltpu

DIM = 96
HEADS = 3
HD = DIM // HEADS
NWIN = 7
P2 = NWIN * NWIN
WS = 8
HW = WS * WS
NPIX = P2 * HW
TOPK = 4
MLPR = 4
SDW = 5
SCALE = DIM ** -0.5
F32 = jnp.float32
BF16 = jnp.bfloat16


def _win_to_img(t):
    return (t.reshape(NWIN, NWIN, WS, WS, DIM)
             .transpose(0, 2, 1, 3, 4)
             .reshape(NWIN * WS, NWIN * WS, DIM))


def _img_to_win(t):
    return (t.reshape(NWIN, WS, NWIN, WS, DIM)
             .transpose(0, 2, 1, 3, 4)
             .reshape(NPIX, DIM))


def _layernorm(xf, g, b):
    mu = jnp.mean(xf, axis=-1, keepdims=True)
    xc = xf - mu
    var = jnp.mean(xc * xc, axis=-1, keepdims=True)
    return xc * jax.lax.rsqrt(var + 1e-6) * g + b


def _bdot(a, b):
    return jnp.dot(a.astype(BF16), b, preferred_element_type=F32)


def _pre_body(xf, g1_ref, b1_ref, wq_ref, wk_ref, wv_ref, bq_ref, bk_ref,
              bv_ref, wl_ref, bl_ref,
              q_ref, k_ref, v_ref, lepe_ref, topi_ref, pad_ref):
    y = _layernorm(xf, g1_ref[...], b1_ref[...])
    q = jnp.dot(y, wq_ref[...], preferred_element_type=F32) + bq_ref[...]
    k = jnp.dot(y, wk_ref[...], preferred_element_type=F32) + bk_ref[...]
    v = jnp.dot(y, wv_ref[...], preferred_element_type=F32) + bv_ref[...]
    q_ref[0] = q.astype(BF16).reshape(P2, HW, DIM)
    k_ref[0] = k.astype(BF16).reshape(P2, HW, DIM)
    v_ref[0] = v.astype(BF16).reshape(P2, HW, DIM)

    pad_ref[...] = jnp.zeros((60, 60, DIM), F32)
    pad_ref[2:58, 2:58, :] = _win_to_img(v)
    acc = jnp.broadcast_to(bl_ref[...], (NPIX, DIM))
    for di in range(SDW):
        for dj in range(SDW):
            sh = pad_ref[di:di + 56, dj:dj + 56, :].reshape(NPIX, DIM)
            acc = acc + sh * wl_ref[di * SDW + dj:di * SDW + dj + 1, :]
    lepe_ref[0] = _img_to_win(acc.reshape(56, 56, DIM)).reshape(NPIX, DIM)

    qm = jnp.mean(q.reshape(P2, HW, DIM), axis=1)
    km = jnp.mean(k.reshape(P2, HW, DIM), axis=1)
    lT = jax.lax.dot_general(km, qm * SCALE, (((1,), (1,)), ((), ())),
                             preferred_element_type=F32)
    iota0 = jax.lax.broadcasted_iota(jnp.int32, (P2, P2), 0)
    for t in range(TOPK):
        mx = jnp.max(lT, axis=0, keepdims=True)
        cand = jnp.where(lT >= mx, iota0, jnp.int32(2 ** 30))
        idx = jnp.min(cand, axis=0, keepdims=True)
        topi_ref[0, t:t + 1, :] = idx
        lT = jnp.where(iota0 == idx, -jnp.inf, lT)


def _attn_body(topi_sm, q_ref, k_ref, v_ref, ao_ref):
    n = pl.program_id(0)
    for p in range(P2):
        q = q_ref[0, p]
        ks = []
        vs = []
        for t in range(TOPK):
            s = topi_sm[n, t, p]
            ks.append(k_ref[0, s])
            vs.append(v_ref[0, s])
        kk = jnp.concatenate(ks, axis=0)
        vv = jnp.concatenate(vs, axis=0)
        outs = []
        for hh in range(HEADS):
            qh = q[:, hh * HD:(hh + 1) * HD]
            kh = kk[:, hh * HD:(hh + 1) * HD]
            s = jax.lax.dot_general(qh, kh, (((1,), (1,)), ((), ())),
                                    preferred_element_type=F32)
            e = jnp.exp(s * SCALE)
            l = jnp.sum(e, axis=1, keepdims=True)
            o = _bdot(e, vv[:, hh * HD:(hh + 1) * HD])
            outs.append(o / l)
        ao_ref[p] = jnp.concatenate(outs, axis=1)


def _post_body(ao, lepe_ref, x_ref, wo_ref, bo_ref, g2_ref, b2_ref,
               w1_ref, bm1_ref, w2_ref, bm2_ref):
    ao = ao + lepe_ref[0]
    x1 = x_ref[0] + _bdot(ao, wo_ref[...]) + bo_ref[...]
    y = _layernorm(x1, g2_ref[...], b2_ref[...])
    t1 = _bdot(y, w1_ref[...]) + bm1_ref[...]
    t1 = t1 * 0.5 * (1.0 + jax.lax.erf(t1 * (2.0 ** -0.5)))
    y2 = _bdot(t1, w2_ref[...]) + bm2_ref[...]
    return x1 + y2


def _pre_kernel(x_ref, *refs):
    _pre_body(x_ref[0], *refs)


def _mid_kernel(topi_sm, q_ref, k_ref, v_ref, lepe_ref, x_ref,
                wo_ref, bo_ref, g2_ref, b2_ref, w1_ref, bm1_ref, w2_ref,
                bm2_ref,
                g1_ref, b1_ref, wq_ref, wk_ref, wv_ref, bq_ref, bk_ref,
                bv_ref, wl_ref, bl_ref,
                xo_ref, q2_ref, k2_ref, v2_ref, lepe2_ref, topi2_ref,
                pad_ref, ao_ref):
    _attn_body(topi_sm, q_ref, k_ref, v_ref, ao_ref)
    x2 = _post_body(ao_ref[...].reshape(NPIX, DIM), lepe_ref, x_ref,
                    wo_ref, bo_ref, g2_ref, b2_ref,
                    w1_ref, bm1_ref, w2_ref, bm2_ref)
    xo_ref[0] = x2
    _pre_body(x2, g1_ref, b1_ref, wq_ref, wk_ref, wv_ref, bq_ref, bk_ref,
              bv_ref, wl_ref, bl_ref,
              q2_ref, k2_ref, v2_ref, lepe2_ref, topi2_ref, pad_ref)


def _tail_kernel(topi_sm, q_ref, k_ref, v_ref, lepe_ref, x_ref,
                 wo_ref, bo_ref, g2_ref, b2_ref, w1_ref, bm1_ref, w2_ref,
                 bm2_ref,
                 cw1_ref, cb1_ref, cw2_ref, cb2_ref, sw_ref, sb_ref,
                 wc_ref, bc_ref, out_ref, pad_ref, ao_ref):
    _attn_body(topi_sm, q_ref, k_ref, v_ref, ao_ref)
    xf = _post_body(ao_ref[...].reshape(NPIX, DIM), lepe_ref, x_ref,
                    wo_ref, bo_ref, g2_ref, b2_ref,
                    w1_ref, bm1_ref, w2_ref, bm2_ref)
    xm = jnp.mean(xf, axis=0, keepdims=True)
    c1 = jax.nn.relu(jnp.dot(xm, cw1_ref[...],
                             preferred_element_type=F32) + cb1_ref[...])
    cse = jax.nn.sigmoid(jnp.dot(c1, cw2_ref[...],
                                 preferred_element_type=F32) + cb2_ref[...])
    sse = jax.nn.sigmoid(_bdot(xf, sw_ref[...]) + sb_ref[...])
    y = xf * (cse + sse)

    pad_ref[0:58, 0:58, :] = jnp.zeros((58, 58, DIM), F32)
    pad_ref[1:57, 1:57, :] = _win_to_img(y)
    acc = jnp.broadcast_to(bc_ref[...], (NPIX, DIM))
    for di in range(3):
        for dj in range(3):
            sh = pad_ref[di:di + 56, dj:dj + 56, :].reshape(NPIX, DIM)
            acc = acc + _bdot(sh, wc_ref[di * 3 + dj])
    out_ref[0] = acc.reshape(56, 56, DIM)


def _full(shape):
    nd = len(shape)
    return pl.BlockSpec(shape, lambda n, *_: (0,) * nd)


def _per_img(shape):
    nd = len(shape)
    return pl.BlockSpec((1,) + shape, lambda n, *_: (n,) + (0,) * nd)


def _row2(a):
    return a.reshape(1, -1)


def _block_weights(p):
    wq = p['Wqkv'][:, :DIM]
    wk = p['Wqkv'][:, DIM:2 * DIM]
    wv = p['Wqkv'][:, 2 * DIM:]
    pre_args = (_row2(p['g1']), _row2(p['b1']), wq, wk, wv,
                _row2(p['bqkv'][:DIM]), _row2(p['bqkv'][DIM:2 * DIM]),
                _row2(p['bqkv'][2 * DIM:]),
                p['Wlepe'].reshape(SDW * SDW, DIM), _row2(p['blepe']))
    post_args = (p['Wo'].astype(BF16), _row2(p['bo']), _row2(p['g2']),
                 _row2(p['b2']), p['W1'].astype(BF16), _row2(p['bm1']),
                 p['W2'].astype(BF16), _row2(p['bm2']))
    return pre_args, post_args


def _pre_out(N):
    specs = ([_per_img((P2, HW, DIM))] * 3 +
             [_per_img((NPIX, DIM)), _per_img((TOPK, P2))])
    shapes = ([jax.ShapeDtypeStruct((N, P2, HW, DIM), BF16)] * 3 +
              [jax.ShapeDtypeStruct((N, NPIX, DIM), F32),
               jax.ShapeDtypeStruct((N, TOPK, P2), jnp.int32)])
    return specs, shapes


@jax.jit
def kernel(x, params):
    N = x.shape[0]
    xw = jnp.transpose(x, (0, 2, 3, 1))
    xw = (xw.reshape(N, NWIN, WS, NWIN, WS, DIM)
            .transpose(0, 1, 3, 2, 4, 5)
            .reshape(N, NPIX, DIM))

    b0, b1 = params['blocks']
    pre0, post0 = _block_weights(b0)
    pre1, post1 = _block_weights(b1)
    pre_specs, pre_shapes = _pre_out(N)

    q, k, v, lepe, topi = pl.pallas_call(
        _pre_kernel,
        grid=(N,),
        in_specs=[_per_img((NPIX, DIM))] + [_full(a.shape) for a in pre0],
        out_specs=pre_specs,
        out_shape=pre_shapes,
        scratch_shapes=[pltpu.VMEM((60, 60, DIM), F32)],
    )(xw, *pre0)

    xw2, q, k, v, lepe, topi = pl.pallas_call(
        _mid_kernel,
        grid_spec=pltpu.PrefetchScalarGridSpec(
            num_scalar_prefetch=1,
            grid=(N,),
            in_specs=[_per_img((P2, HW, DIM))] * 3 +
                     [_per_img((NPIX, DIM))] * 2 +
                     [_full(a.shape) for a in post0 + pre1],
            out_specs=[_per_img((NPIX, DIM))] + pre_specs,
            scratch_shapes=[pltpu.VMEM((60, 60, DIM), F32),
                            pltpu.VMEM((P2, HW, DIM), F32)],
        ),
        out_shape=[jax.ShapeDtypeStruct((N, NPIX, DIM), F32)] + pre_shapes,
    )(topi, q, k, v, lepe, xw, *(post0 + pre1))

    tail_w = (params['cse_w1'], _row2(params['cse_b1']),
              params['cse_w2'], _row2(params['cse_b2']),
              params['sse_w'].astype(BF16), _row2(params['sse_b']),
              params['conv_w'].reshape(9, DIM, DIM).astype(BF16),
              _row2(params['conv_b']))
    out = pl.pallas_call(
        _tail_kernel,
        grid_spec=pltpu.PrefetchScalarGridSpec(
            num_scalar_prefetch=1,
            grid=(N,),
            in_specs=[_per_img((P2, HW, DIM))] * 3 +
                     [_per_img((NPIX, DIM))] * 2 +
                     [_full(a.shape) for a in post1 + tail_w],
            out_specs=_per_img((56, 56, DIM)),
            scratch_shapes=[pltpu.VMEM((58, 58, DIM), F32),
                            pltpu.VMEM((P2, HW, DIM), F32)],
        ),
        out_shape=jax.ShapeDtypeStruct((N, 56, 56, DIM), F32),
    )(topi, q, k, v, lepe, xw2, *(post1 + tail_w))

    return jnp.transpose(out, (0, 3, 1, 2))


# speedup vs baseline: 2.9937x; 1.0034x over previous
import functools

import jax
import jax.numpy as jnp
import numpy as np
from jax.experimental import pallas as pl
from jax.experimental.pallas import tpu as pltpu

DIM = 96
HEADS = 3
HD = DIM // HEADS
NWIN = 7
P2 = NWIN * NWIN
WS = 8
HW = WS * WS
NPIX = P2 * HW
TOPK = 4
MLPR = 4
SDW = 5
SCALE = DIM ** -0.5
F32 = jnp.float32
BF16 = jnp.bfloat16


def _win_to_img(t):
    return (t.reshape(NWIN, NWIN, WS, WS, DIM)
             .transpose(0, 2, 1, 3, 4)
             .reshape(NWIN * WS, NWIN * WS, DIM))


def _img_to_win(t):
    return (t.reshape(NWIN, WS, NWIN, WS, DIM)
             .transpose(0, 2, 1, 3, 4)
             .reshape(NPIX, DIM))


def _layernorm(xf, g, b):
    mu = jnp.mean(xf, axis=-1, keepdims=True)
    xc = xf - mu
    var = jnp.mean(xc * xc, axis=-1, keepdims=True)
    return xc * jax.lax.rsqrt(var + 1e-6) * g + b


def _bdot(a, b):
    return jnp.dot(a.astype(BF16), b, preferred_element_type=F32)


def _pre_body(xf, g1_ref, b1_ref, wq_ref, wk_ref, wv_ref, bq_ref, bk_ref,
              bv_ref, wl_ref, bl_ref,
              q_ref, k_ref, v_ref, lepe_ref, topi_ref, pad_ref):
    y = _layernorm(xf, g1_ref[...], b1_ref[...])
    q = jnp.dot(y, wq_ref[...], preferred_element_type=F32) + bq_ref[...]
    k = jnp.dot(y, wk_ref[...], preferred_element_type=F32) + bk_ref[...]
    v = jnp.dot(y, wv_ref[...], preferred_element_type=F32) + bv_ref[...]
    q_ref[0] = (q * SCALE).astype(BF16).reshape(P2, HW, DIM)
    k_ref[0] = k.astype(BF16).reshape(P2, HW, DIM)
    v_ref[0] = v.astype(BF16).reshape(P2, HW, DIM)

    pad_ref[...] = jnp.zeros((60, 60, DIM), F32)
    pad_ref[2:58, 2:58, :] = _win_to_img(v)
    acc = jnp.broadcast_to(bl_ref[...], (NPIX, DIM))
    for di in range(SDW):
        for dj in range(SDW):
            sh = pad_ref[di:di + 56, dj:dj + 56, :].reshape(NPIX, DIM)
            acc = acc + sh * wl_ref[di * SDW + dj:di * SDW + dj + 1, :]
    lepe_ref[0] = _img_to_win(acc.reshape(56, 56, DIM)).reshape(NPIX, DIM)

    qm = jnp.mean(q.reshape(P2, HW, DIM), axis=1)
    km = jnp.mean(k.reshape(P2, HW, DIM), axis=1)
    lT = jax.lax.dot_general(km, qm * SCALE, (((1,), (1,)), ((), ())),
                             preferred_element_type=F32)
    iota0 = jax.lax.broadcasted_iota(jnp.int32, (P2, P2), 0)
    for t in range(TOPK):
        mx = jnp.max(lT, axis=0, keepdims=True)
        cand = jnp.where(lT >= mx, iota0, jnp.int32(2 ** 30))
        idx = jnp.min(cand, axis=0, keepdims=True)
        topi_ref[0, t:t + 1, :] = idx
        lT = jnp.where(iota0 == idx, -jnp.inf, lT)


def _attn_body(topi_sm, q_ref, k_ref, v_ref, ao_ref):
    n = pl.program_id(0)
    for p in range(P2):
        q = q_ref[0, p]
        ks = []
        vs = []
        for t in range(TOPK):
            s = topi_sm[n, t, p]
            ks.append(k_ref[0, s])
            vs.append(v_ref[0, s])
        kk = jnp.concatenate(ks, axis=0)
        vv = jnp.concatenate(vs, axis=0)
        outs = []
        for hh in range(HEADS):
            qh = q[:, hh * HD:(hh + 1) * HD]
            kh = kk[:, hh * HD:(hh + 1) * HD]
            s = jax.lax.dot_general(qh, kh, (((1,), (1,)), ((), ())),
                                    preferred_element_type=F32)
            e = jnp.exp(s)
            l = jnp.sum(e, axis=1, keepdims=True)
            o = _bdot(e, vv[:, hh * HD:(hh + 1) * HD])
            outs.append(o / l)
        ao_ref[p] = jnp.concatenate(outs, axis=1)


def _post_body(ao, lepe_ref, x_ref, wo_ref, bo_ref, g2_ref, b2_ref,
               w1_ref, bm1_ref, w2_ref, bm2_ref):
    ao = ao + lepe_ref[0]
    x1 = x_ref[0] + _bdot(ao, wo_ref[...]) + bo_ref[...]
    y = _layernorm(x1, g2_ref[...], b2_ref[...])
    t1 = _bdot(y, w1_ref[...]) + bm1_ref[...]
    t1 = t1 * 0.5 * (1.0 + jax.lax.erf(t1 * (2.0 ** -0.5)))
    y2 = _bdot(t1, w2_ref[...]) + bm2_ref[...]
    return x1 + y2


def _pre_kernel(x_ref, *refs):
    _pre_body(x_ref[0], *refs)


def _mid_kernel(topi_sm, q_ref, k_ref, v_ref, lepe_ref, x_ref,
                wo_ref, bo_ref, g2_ref, b2_ref, w1_ref, bm1_ref, w2_ref,
                bm2_ref,
                g1_ref, b1_ref, wq_ref, wk_ref, wv_ref, bq_ref, bk_ref,
                bv_ref, wl_ref, bl_ref,
                xo_ref, q2_ref, k2_ref, v2_ref, lepe2_ref, topi2_ref,
                pad_ref, ao_ref):
    _attn_body(topi_sm, q_ref, k_ref, v_ref, ao_ref)
    x2 = _post_body(ao_ref[...].reshape(NPIX, DIM), lepe_ref, x_ref,
                    wo_ref, bo_ref, g2_ref, b2_ref,
                    w1_ref, bm1_ref, w2_ref, bm2_ref)
    xo_ref[0] = x2
    _pre_body(x2, g1_ref, b1_ref, wq_ref, wk_ref, wv_ref, bq_ref, bk_ref,
              bv_ref, wl_ref, bl_ref,
              q2_ref, k2_ref, v2_ref, lepe2_ref, topi2_ref, pad_ref)


def _tail_kernel(topi_sm, q_ref, k_ref, v_ref, lepe_ref, x_ref,
                 wo_ref, bo_ref, g2_ref, b2_ref, w1_ref, bm1_ref, w2_ref,
                 bm2_ref,
                 cw1_ref, cb1_ref, cw2_ref, cb2_ref, sw_ref, sb_ref,
                 wc_ref, bc_ref, out_ref, pad_ref, ao_ref):
    _attn_body(topi_sm, q_ref, k_ref, v_ref, ao_ref)
    xf = _post_body(ao_ref[...].reshape(NPIX, DIM), lepe_ref, x_ref,
                    wo_ref, bo_ref, g2_ref, b2_ref,
                    w1_ref, bm1_ref, w2_ref, bm2_ref)
    xm = jnp.mean(xf, axis=0, keepdims=True)
    c1 = jax.nn.relu(jnp.dot(xm, cw1_ref[...],
                             preferred_element_type=F32) + cb1_ref[...])
    cse = jax.nn.sigmoid(jnp.dot(c1, cw2_ref[...],
                                 preferred_element_type=F32) + cb2_ref[...])
    sse = jax.nn.sigmoid(_bdot(xf, sw_ref[...]) + sb_ref[...])
    y = xf * (cse + sse)

    pad_ref[0:58, 0:58, :] = jnp.zeros((58, 58, DIM), F32)
    pad_ref[1:57, 1:57, :] = _win_to_img(y)
    acc = jnp.broadcast_to(bc_ref[...], (NPIX, DIM))
    for di in range(3):
        for dj in range(3):
            sh = pad_ref[di:di + 56, dj:dj + 56, :].reshape(NPIX, DIM)
            acc = acc + _bdot(sh, wc_ref[di * 3 + dj])
    out_ref[0] = acc.reshape(56, 56, DIM)


def _full(shape):
    nd = len(shape)
    return pl.BlockSpec(shape, lambda n, *_: (0,) * nd)


def _per_img(shape):
    nd = len(shape)
    return pl.BlockSpec((1,) + shape, lambda n, *_: (n,) + (0,) * nd)


def _row2(a):
    return a.reshape(1, -1)


def _block_weights(p):
    wq = p['Wqkv'][:, :DIM]
    wk = p['Wqkv'][:, DIM:2 * DIM]
    wv = p['Wqkv'][:, 2 * DIM:]
    pre_args = (_row2(p['g1']), _row2(p['b1']), wq, wk, wv,
                _row2(p['bqkv'][:DIM]), _row2(p['bqkv'][DIM:2 * DIM]),
                _row2(p['bqkv'][2 * DIM:]),
                p['Wlepe'].reshape(SDW * SDW, DIM), _row2(p['blepe']))
    post_args = (p['Wo'].astype(BF16), _row2(p['bo']), _row2(p['g2']),
                 _row2(p['b2']), p['W1'].astype(BF16), _row2(p['bm1']),
                 p['W2'].astype(BF16), _row2(p['bm2']))
    return pre_args, post_args


def _pre_out(N):
    specs = ([_per_img((P2, HW, DIM))] * 3 +
             [_per_img((NPIX, DIM)), _per_img((TOPK, P2))])
    shapes = ([jax.ShapeDtypeStruct((N, P2, HW, DIM), BF16)] * 3 +
              [jax.ShapeDtypeStruct((N, NPIX, DIM), F32),
               jax.ShapeDtypeStruct((N, TOPK, P2), jnp.int32)])
    return specs, shapes


@jax.jit
def kernel(x, params):
    N = x.shape[0]
    xw = jnp.transpose(x, (0, 2, 3, 1))
    xw = (xw.reshape(N, NWIN, WS, NWIN, WS, DIM)
            .transpose(0, 1, 3, 2, 4, 5)
            .reshape(N, NPIX, DIM))

    b0, b1 = params['blocks']
    pre0, post0 = _block_weights(b0)
    pre1, post1 = _block_weights(b1)
    pre_specs, pre_shapes = _pre_out(N)

    q, k, v, lepe, topi = pl.pallas_call(
        _pre_kernel,
        grid=(N,),
        in_specs=[_per_img((NPIX, DIM))] + [_full(a.shape) for a in pre0],
        out_specs=pre_specs,
        out_shape=pre_shapes,
        scratch_shapes=[pltpu.VMEM((60, 60, DIM), F32)],
    )(xw, *pre0)

    xw2, q, k, v, lepe, topi = pl.pallas_call(
        _mid_kernel,
        grid_spec=pltpu.PrefetchScalarGridSpec(
            num_scalar_prefetch=1,
            grid=(N,),
            in_specs=[_per_img((P2, HW, DIM))] * 3 +
                     [_per_img((NPIX, DIM))] * 2 +
                     [_full(a.shape) for a in post0 + pre1],
            out_specs=[_per_img((NPIX, DIM))] + pre_specs,
            scratch_shapes=[pltpu.VMEM((60, 60, DIM), F32),
                            pltpu.VMEM((P2, HW, DIM), F32)],
        ),
        out_shape=[jax.ShapeDtypeStruct((N, NPIX, DIM), F32)] + pre_shapes,
    )(topi, q, k, v, lepe, xw, *(post0 + pre1))

    tail_w = (params['cse_w1'], _row2(params['cse_b1']),
              params['cse_w2'], _row2(params['cse_b2']),
              params['sse_w'].astype(BF16), _row2(params['sse_b']),
              params['conv_w'].reshape(9, DIM, DIM).astype(BF16),
              _row2(params['conv_b']))
    out = pl.pallas_call(
        _tail_kernel,
        grid_spec=pltpu.PrefetchScalarGridSpec(
            num_scalar_prefetch=1,
            grid=(N,),
            in_specs=[_per_img((P2, HW, DIM))] * 3 +
                     [_per_img((NPIX, DIM))] * 2 +
                     [_full(a.shape) for a in post1 + tail_w],
            out_specs=_per_img((56, 56, DIM)),
            scratch_shapes=[pltpu.VMEM((58, 58, DIM), F32),
                            pltpu.VMEM((P2, HW, DIM), F32)],
        ),
        out_shape=jax.ShapeDtypeStruct((N, 56, 56, DIM), F32),
    )(topi, q, k, v, lepe, xw2, *(post1 + tail_w))

    return jnp.transpose(out, (0, 3, 1, 2))


# speedup vs baseline: 3.4242x; 1.1438x over previous
import functools

import jax
import jax.numpy as jnp
import numpy as np
from jax.experimental import pallas as pl
from jax.experimental.pallas import tpu as pltpu

DIM = 96
HEADS = 3
HD = DIM // HEADS
NWIN = 7
P2 = NWIN * NWIN
WS = 8
HW = WS * WS
NPIX = P2 * HW
TOPK = 4
MLPR = 4
SDW = 5
SCALE = DIM ** -0.5
F32 = jnp.float32
BF16 = jnp.bfloat16


def _win_to_img(t):
    return (t.reshape(NWIN, NWIN, WS, WS, DIM)
             .transpose(0, 2, 1, 3, 4)
             .reshape(NWIN * WS, NWIN * WS, DIM))


def _img_to_win(t):
    return (t.reshape(NWIN, WS, NWIN, WS, DIM)
             .transpose(0, 2, 1, 3, 4)
             .reshape(NPIX, DIM))


def _layernorm(xf, g, b):
    mu = jnp.mean(xf, axis=-1, keepdims=True)
    xc = xf - mu
    var = jnp.mean(xc * xc, axis=-1, keepdims=True)
    return xc * jax.lax.rsqrt(var + 1e-6) * g + b


def _bdot(a, b):
    return jnp.dot(a.astype(BF16), b, preferred_element_type=F32)


def _pre_body(xf, g1_ref, b1_ref, wq_ref, wk_ref, wv_ref, bq_ref, bk_ref,
              bv_ref, wl_ref, bl_ref,
              q_ref, k_ref, v_ref, lepe_ref, topi_ref, pad_ref):
    y = _layernorm(xf, g1_ref[...], b1_ref[...])
    q = jnp.dot(y, wq_ref[...], preferred_element_type=F32) + bq_ref[...]
    k = jnp.dot(y, wk_ref[...], preferred_element_type=F32) + bk_ref[...]
    v = jnp.dot(y, wv_ref[...], preferred_element_type=F32) + bv_ref[...]
    q_ref[0] = (q * SCALE).astype(BF16).reshape(P2, HW, DIM)
    k_ref[0] = k.astype(BF16).reshape(P2, HW, DIM)
    v_ref[0] = v.astype(BF16).reshape(P2, HW, DIM)

    pad_ref[...] = jnp.zeros((60, 60, DIM), F32)
    pad_ref[2:58, 2:58, :] = _win_to_img(v)
    acc = jnp.broadcast_to(bl_ref[...], (NPIX, DIM))
    for di in range(SDW):
        for dj in range(SDW):
            sh = pad_ref[di:di + 56, dj:dj + 56, :].reshape(NPIX, DIM)
            acc = acc + sh * wl_ref[di * SDW + dj:di * SDW + dj + 1, :]
    lepe_ref[0] = _img_to_win(acc.reshape(56, 56, DIM)).reshape(NPIX, DIM)

    qm = jnp.mean(q.reshape(P2, HW, DIM), axis=1)
    km = jnp.mean(k.reshape(P2, HW, DIM), axis=1)
    lT = jax.lax.dot_general(km, qm * SCALE, (((1,), (1,)), ((), ())),
                             preferred_element_type=F32)
    iota0 = jax.lax.broadcasted_iota(jnp.int32, (P2, P2), 0)
    for t in range(TOPK):
        mx = jnp.max(lT, axis=0, keepdims=True)
        cand = jnp.where(lT >= mx, iota0, jnp.int32(2 ** 30))
        idx = jnp.min(cand, axis=0, keepdims=True)
        topi_ref[0, t:t + 1, :] = idx
        lT = jnp.where(iota0 == idx, -jnp.inf, lT)


def _attn_body(topi_sm, q_ref, k_ref, v_ref, ao_ref):
    n = pl.program_id(0)
    for p in range(P2):
        q = q_ref[0, p]
        qs = [q[:, hh * HD:(hh + 1) * HD] for hh in range(HEADS)]
        o = [jnp.zeros((HW, HD), F32) for _ in range(HEADS)]
        l = [jnp.zeros((HW, 1), F32) for _ in range(HEADS)]
        for t in range(TOPK):
            s = topi_sm[n, t, p]
            kt = k_ref[0, s]
            vt = v_ref[0, s]
            for hh in range(HEADS):
                kh = kt[:, hh * HD:(hh + 1) * HD]
                sc = jax.lax.dot_general(qs[hh], kh, (((1,), (1,)), ((), ())),
                                         preferred_element_type=F32)
                e = jnp.exp(sc)
                l[hh] = l[hh] + jnp.sum(e, axis=1, keepdims=True)
                o[hh] = o[hh] + _bdot(e, vt[:, hh * HD:(hh + 1) * HD])
        ao_ref[p] = jnp.concatenate([o[hh] / l[hh] for hh in range(HEADS)],
                                    axis=1)


def _post_body(ao, lepe_ref, x_ref, wo_ref, bo_ref, g2_ref, b2_ref,
               w1_ref, bm1_ref, w2_ref, bm2_ref):
    ao = ao + lepe_ref[0]
    x1 = x_ref[0] + _bdot(ao, wo_ref[...]) + bo_ref[...]
    y = _layernorm(x1, g2_ref[...], b2_ref[...])
    t1 = _bdot(y, w1_ref[...]) + bm1_ref[...]
    t1 = t1 * 0.5 * (1.0 + jax.lax.erf(t1 * (2.0 ** -0.5)))
    y2 = _bdot(t1, w2_ref[...]) + bm2_ref[...]
    return x1 + y2


def _pre_kernel(x_ref, *refs):
    _pre_body(x_ref[0], *refs)


def _mid_kernel(topi_sm, q_ref, k_ref, v_ref, lepe_ref, x_ref,
                wo_ref, bo_ref, g2_ref, b2_ref, w1_ref, bm1_ref, w2_ref,
                bm2_ref,
                g1_ref, b1_ref, wq_ref, wk_ref, wv_ref, bq_ref, bk_ref,
                bv_ref, wl_ref, bl_ref,
                xo_ref, q2_ref, k2_ref, v2_ref, lepe2_ref, topi2_ref,
                pad_ref, ao_ref):
    _attn_body(topi_sm, q_ref, k_ref, v_ref, ao_ref)
    x2 = _post_body(ao_ref[...].reshape(NPIX, DIM), lepe_ref, x_ref,
                    wo_ref, bo_ref, g2_ref, b2_ref,
                    w1_ref, bm1_ref, w2_ref, bm2_ref)
    xo_ref[0] = x2
    _pre_body(x2, g1_ref, b1_ref, wq_ref, wk_ref, wv_ref, bq_ref, bk_ref,
              bv_ref, wl_ref, bl_ref,
              q2_ref, k2_ref, v2_ref, lepe2_ref, topi2_ref, pad_ref)


def _tail_kernel(topi_sm, q_ref, k_ref, v_ref, lepe_ref, x_ref,
                 wo_ref, bo_ref, g2_ref, b2_ref, w1_ref, bm1_ref, w2_ref,
                 bm2_ref,
                 cw1_ref, cb1_ref, cw2_ref, cb2_ref, sw_ref, sb_ref,
                 wc_ref, bc_ref, out_ref, pad_ref, ao_ref):
    _attn_body(topi_sm, q_ref, k_ref, v_ref, ao_ref)
    xf = _post_body(ao_ref[...].reshape(NPIX, DIM), lepe_ref, x_ref,
                    wo_ref, bo_ref, g2_ref, b2_ref,
                    w1_ref, bm1_ref, w2_ref, bm2_ref)
    xm = jnp.mean(xf, axis=0, keepdims=True)
    c1 = jax.nn.relu(jnp.dot(xm, cw1_ref[...],
                             preferred_element_type=F32) + cb1_ref[...])
    cse = jax.nn.sigmoid(jnp.dot(c1, cw2_ref[...],
                                 preferred_element_type=F32) + cb2_ref[...])
    sse = jax.nn.sigmoid(_bdot(xf, sw_ref[...]) + sb_ref[...])
    y = xf * (cse + sse)

    pad_ref[0:58, 0:58, :] = jnp.zeros((58, 58, DIM), F32)
    pad_ref[1:57, 1:57, :] = _win_to_img(y)
    acc = jnp.broadcast_to(bc_ref[...], (NPIX, DIM))
    for di in range(3):
        for dj in range(3):
            sh = pad_ref[di:di + 56, dj:dj + 56, :].reshape(NPIX, DIM)
            acc = acc + _bdot(sh, wc_ref[di * 3 + dj])
    out_ref[0] = acc.reshape(56, 56, DIM)


def _full(shape):
    nd = len(shape)
    return pl.BlockSpec(shape, lambda n, *_: (0,) * nd)


def _per_img(shape):
    nd = len(shape)
    return pl.BlockSpec((1,) + shape, lambda n, *_: (n,) + (0,) * nd)


def _row2(a):
    return a.reshape(1, -1)


def _block_weights(p):
    wq = p['Wqkv'][:, :DIM]
    wk = p['Wqkv'][:, DIM:2 * DIM]
    wv = p['Wqkv'][:, 2 * DIM:]
    pre_args = (_row2(p['g1']), _row2(p['b1']), wq, wk, wv,
                _row2(p['bqkv'][:DIM]), _row2(p['bqkv'][DIM:2 * DIM]),
                _row2(p['bqkv'][2 * DIM:]),
                p['Wlepe'].reshape(SDW * SDW, DIM), _row2(p['blepe']))
    post_args = (p['Wo'].astype(BF16), _row2(p['bo']), _row2(p['g2']),
                 _row2(p['b2']), p['W1'].astype(BF16), _row2(p['bm1']),
                 p['W2'].astype(BF16), _row2(p['bm2']))
    return pre_args, post_args


def _pre_out(N):
    specs = ([_per_img((P2, HW, DIM))] * 3 +
             [_per_img((NPIX, DIM)), _per_img((TOPK, P2))])
    shapes = ([jax.ShapeDtypeStruct((N, P2, HW, DIM), BF16)] * 3 +
              [jax.ShapeDtypeStruct((N, NPIX, DIM), F32),
               jax.ShapeDtypeStruct((N, TOPK, P2), jnp.int32)])
    return specs, shapes


@jax.jit
def kernel(x, params):
    N = x.shape[0]
    xw = jnp.transpose(x, (0, 2, 3, 1))
    xw = (xw.reshape(N, NWIN, WS, NWIN, WS, DIM)
            .transpose(0, 1, 3, 2, 4, 5)
            .reshape(N, NPIX, DIM))

    b0, b1 = params['blocks']
    pre0, post0 = _block_weights(b0)
    pre1, post1 = _block_weights(b1)
    pre_specs, pre_shapes = _pre_out(N)

    q, k, v, lepe, topi = pl.pallas_call(
        _pre_kernel,
        grid=(N,),
        in_specs=[_per_img((NPIX, DIM))] + [_full(a.shape) for a in pre0],
        out_specs=pre_specs,
        out_shape=pre_shapes,
        scratch_shapes=[pltpu.VMEM((60, 60, DIM), F32)],
    )(xw, *pre0)

    xw2, q, k, v, lepe, topi = pl.pallas_call(
        _mid_kernel,
        grid_spec=pltpu.PrefetchScalarGridSpec(
            num_scalar_prefetch=1,
            grid=(N,),
            in_specs=[_per_img((P2, HW, DIM))] * 3 +
                     [_per_img((NPIX, DIM))] * 2 +
                     [_full(a.shape) for a in post0 + pre1],
            out_specs=[_per_img((NPIX, DIM))] + pre_specs,
            scratch_shapes=[pltpu.VMEM((60, 60, DIM), F32),
                            pltpu.VMEM((P2, HW, DIM), F32)],
        ),
        out_shape=[jax.ShapeDtypeStruct((N, NPIX, DIM), F32)] + pre_shapes,
    )(topi, q, k, v, lepe, xw, *(post0 + pre1))

    tail_w = (params['cse_w1'], _row2(params['cse_b1']),
              params['cse_w2'], _row2(params['cse_b2']),
              params['sse_w'].astype(BF16), _row2(params['sse_b']),
              params['conv_w'].reshape(9, DIM, DIM).astype(BF16),
              _row2(params['conv_b']))
    out = pl.pallas_call(
        _tail_kernel,
        grid_spec=pltpu.PrefetchScalarGridSpec(
            num_scalar_prefetch=1,
            grid=(N,),
            in_specs=[_per_img((P2, HW, DIM))] * 3 +
                     [_per_img((NPIX, DIM))] * 2 +
                     [_full(a.shape) for a in post1 + tail_w],
            out_specs=_per_img((56, 56, DIM)),
            scratch_shapes=[pltpu.VMEM((58, 58, DIM), F32),
                            pltpu.VMEM((P2, HW, DIM), F32)],
        ),
        out_shape=jax.ShapeDtypeStruct((N, 56, 56, DIM), F32),
    )(topi, q, k, v, lepe, xw2, *(post1 + tail_w))

    return jnp.transpose(out, (0, 3, 1, 2))
